# Initial kernel scaffold; baseline (speedup 1.0000x reference)
import jax
import jax.numpy as jnp
from jax.experimental import pallas as pl


def kernel(ids, table):
    raise NotImplementedError("write your pallas kernel here")


# speedup vs baseline: 2.9656x; 2.9656x over previous
import functools
import jax
import jax.numpy as jnp
from jax import lax
from jax.experimental import pallas as pl
from jax.experimental.pallas import tpu as pltpu, tpu_sc as plsc

_D = 128
_NW = 32
_CHUNK = 128


def _make_gather(B):
    b_per_w = B // _NW
    n_chunks = b_per_w // _CHUNK
    mesh = plsc.VectorSubcoreMesh(core_axis_name="c", subcore_axis_name="s")

    @functools.partial(
        pl.kernel,
        out_type=jax.ShapeDtypeStruct((B, _D), jnp.float32),
        mesh=mesh,
        scratch_types=[
            pltpu.VMEM((b_per_w,), jnp.int32),
            pltpu.VMEM((_CHUNK, _D), jnp.float32),
            pltpu.SemaphoreType.DMA,
        ],
    )
    def gather_kernel(ids_hbm, table_hbm, out_hbm, idx_v, rows_v, sem):
        wid = lax.axis_index("s") * 2 + lax.axis_index("c")
        base = wid * b_per_w
        pltpu.sync_copy(ids_hbm.at[pl.ds(base, b_per_w)], idx_v)

        def step(j, carry):
            off = j * _CHUNK
            pltpu.async_copy(
                table_hbm.at[idx_v.at[pl.ds(off, _CHUNK)]], rows_v, sem
            ).wait()
            pltpu.sync_copy(rows_v, out_hbm.at[pl.ds(base + off, _CHUNK)])
            return carry

        lax.fori_loop(0, n_chunks, step, 0)

    return gather_kernel


def kernel(ids, table):
    batch, hist = ids.shape
    flat_ids = ids.reshape(batch * hist).astype(jnp.int32)
    out = _make_gather(batch * hist)(flat_ids, table)
    return out.reshape(batch, hist, _D)


# speedup vs baseline: 3.3400x; 1.1262x over previous
import functools
import jax
import jax.numpy as jnp
from jax import lax
from jax.experimental import pallas as pl
from jax.experimental.pallas import tpu as pltpu, tpu_sc as plsc

_D = 128
_NW = 32
_CHUNK = 128
_NBUF = 5


def _make_gather(B):
    b_per_w = B // _NW
    n_chunks = b_per_w // _CHUNK
    n_outer = n_chunks // _NBUF
    mesh = plsc.VectorSubcoreMesh(core_axis_name="c", subcore_axis_name="s")

    scratch = (
        [pltpu.VMEM((b_per_w,), jnp.int32)]
        + [pltpu.VMEM((_CHUNK, _D), jnp.float32) for _ in range(_NBUF)]
        + [pltpu.SemaphoreType.DMA for _ in range(2 * _NBUF)]
    )

    @functools.partial(
        pl.kernel,
        out_type=jax.ShapeDtypeStruct((B, _D), jnp.float32),
        mesh=mesh,
        scratch_types=scratch,
    )
    def gather_kernel(ids_hbm, table_hbm, out_hbm, idx_v, *rest):
        bufs = rest[:_NBUF]
        g_sems = rest[_NBUF : 2 * _NBUF]
        o_sems = rest[2 * _NBUF :]
        wid = lax.axis_index("s") * 2 + lax.axis_index("c")
        base = wid * b_per_w
        pltpu.sync_copy(ids_hbm.at[pl.ds(base, b_per_w)], idx_v)

        def g_start(chunk, b):
            pltpu.async_copy(
                table_hbm.at[idx_v.at[pl.ds(chunk * _CHUNK, _CHUNK)]],
                bufs[b],
                g_sems[b],
            )

        def g_wait(b):
            pltpu.make_async_copy(
                table_hbm.at[idx_v.at[pl.ds(0, _CHUNK)]], bufs[b], g_sems[b]
            ).wait()

        def o_start(chunk, b):
            pltpu.async_copy(
                bufs[b],
                out_hbm.at[pl.ds(base + chunk * _CHUNK, _CHUNK)],
                o_sems[b],
            )

        def o_wait(b):
            pltpu.make_async_copy(
                bufs[b], out_hbm.at[pl.ds(base, _CHUNK)], o_sems[b]
            ).wait()

        for b in range(_NBUF - 1):
            g_start(b, b)

        def outer(o, carry):
            for b in range(_NBUF):
                j = o * _NBUF + b
                g_wait(b)
                o_start(j, b)
                pb = (b - 1) % _NBUF
                nxt = j + _NBUF - 1
                if b == 0:
                    @pl.when(o > 0)
                    def _():
                        o_wait(pb)

                    g_start(nxt, pb)
                else:
                    o_wait(pb)

                    @pl.when(o < n_outer - 1)
                    def _():
                        g_start(nxt, pb)

            return carry

        lax.fori_loop(0, n_outer, outer, 0)

        o_wait(_NBUF - 1)

    return gather_kernel


def kernel(ids, table):
    batch, hist = ids.shape
    flat_ids = ids.reshape(batch * hist).astype(jnp.int32)
    out = _make_gather(batch * hist)(flat_ids, table)
    return out.reshape(batch, hist, _D)


# speedup vs baseline: 10.4285x; 3.1223x over previous
import functools
import jax
import jax.numpy as jnp
from jax import lax
from jax.experimental import pallas as pl
from jax.experimental.pallas import tpu as pltpu, tpu_sc as plsc

_D = 128
_NW = 32
_CHUNK = 128
_NBUF = 5


def _make_gather(B):
    b_per_w = B // _NW
    n_chunks = b_per_w // _CHUNK
    n_outer = n_chunks // _NBUF
    mesh = plsc.VectorSubcoreMesh(core_axis_name="c", subcore_axis_name="s")

    scratch = (
        [pltpu.VMEM((b_per_w,), jnp.int32)]
        + [pltpu.VMEM((_CHUNK, _D), jnp.float32) for _ in range(_NBUF)]
        + [pltpu.SemaphoreType.DMA for _ in range(2 * _NBUF)]
    )

    @functools.partial(
        pl.kernel,
        out_type=jax.ShapeDtypeStruct((B, _D), jnp.float32),
        mesh=mesh,
        scratch_types=scratch,
    )
    def gather_kernel(ids_hbm, table_hbm, out_hbm, idx_v, *rest):
        bufs = rest[:_NBUF]
        g_sems = rest[_NBUF : 2 * _NBUF]
        o_sems = rest[2 * _NBUF :]
        wid = lax.axis_index("s") * 2 + lax.axis_index("c")
        base = wid * b_per_w
        pltpu.sync_copy(ids_hbm.at[pl.ds(base, b_per_w)], idx_v)

        def g_start(chunk, b):
            pltpu.async_copy(
                table_hbm.at[idx_v.at[pl.ds(chunk * _CHUNK, _CHUNK)]],
                bufs[b],
                g_sems[b],
            )

        def g_wait(b):
            pltpu.make_async_copy(
                table_hbm.at[idx_v.at[pl.ds(0, _CHUNK)]], bufs[b], g_sems[b]
            ).wait()

        def o_start(chunk, b):
            pltpu.async_copy(
                bufs[b],
                out_hbm.at[pl.ds(base + chunk * _CHUNK, _CHUNK)],
                o_sems[b],
            )

        def o_wait(b):
            pltpu.make_async_copy(
                bufs[b], out_hbm.at[pl.ds(base, _CHUNK)], o_sems[b]
            ).wait()

        for b in range(_NBUF - 1):
            g_start(b, b)

        def outer(o, carry):
            for b in range(_NBUF):
                j = o * _NBUF + b
                g_wait(b)
                o_start(j, b)
                pb = (b - 1) % _NBUF
                nxt = j + _NBUF - 1
                if b == 0:
                    @pl.when(o > 0)
                    def _():
                        o_wait(pb)

                    g_start(nxt, pb)
                else:
                    o_wait(pb)

                    @pl.when(o < n_outer - 1)
                    def _():
                        g_start(nxt, pb)

            return carry

        lax.fori_loop(0, n_outer, outer, 0)

        o_wait(_NBUF - 1)

    return gather_kernel


def kernel(ids, table):
    batch, hist = ids.shape
    flat_ids = ids.T.reshape(batch * hist).astype(jnp.int32)
    out = _make_gather(batch * hist)(flat_ids, table)
    return out.reshape(hist, batch, _D).transpose(1, 0, 2)


# speedup vs baseline: 10.7698x; 1.0327x over previous
import functools
import jax
import jax.numpy as jnp
from jax import lax
from jax.experimental import pallas as pl
from jax.experimental.pallas import tpu as pltpu, tpu_sc as plsc

_D = 128
_NW = 32
_CHUNK = 128
_NBUF = 5


def _make_gather(batch, hist):
    n_chunks = hist
    n_outer = n_chunks // _NBUF
    cols = batch // _NW
    assert cols == _CHUNK and n_outer * _NBUF == n_chunks
    mesh = plsc.VectorSubcoreMesh(core_axis_name="c", subcore_axis_name="s")

    scratch = (
        [pltpu.VMEM((hist, cols), jnp.int32)]
        + [pltpu.VMEM((_CHUNK, _D), jnp.float32) for _ in range(_NBUF)]
        + [pltpu.SemaphoreType.DMA for _ in range(2 * _NBUF)]
    )

    @functools.partial(
        pl.kernel,
        out_type=jax.ShapeDtypeStruct((hist * batch, _D), jnp.float32),
        mesh=mesh,
        scratch_types=scratch,
    )
    def gather_kernel(ids_hbm, table_hbm, out_hbm, idx_v, *rest):
        bufs = rest[:_NBUF]
        g_sems = rest[_NBUF : 2 * _NBUF]
        o_sems = rest[2 * _NBUF :]
        wid = lax.axis_index("s") * 2 + lax.axis_index("c")
        col0 = wid * cols
        pltpu.sync_copy(ids_hbm.at[:, pl.ds(col0, cols)], idx_v)

        def g_start(h, b):
            pltpu.async_copy(table_hbm.at[idx_v.at[h]], bufs[b], g_sems[b])

        def g_wait(b):
            pltpu.make_async_copy(
                table_hbm.at[idx_v.at[0]], bufs[b], g_sems[b]
            ).wait()

        def o_start(h, b):
            pltpu.async_copy(
                bufs[b],
                out_hbm.at[pl.ds(h * batch + col0, _CHUNK)],
                o_sems[b],
            )

        def o_wait(b):
            pltpu.make_async_copy(
                bufs[b], out_hbm.at[pl.ds(col0, _CHUNK)], o_sems[b]
            ).wait()

        for b in range(_NBUF - 1):
            g_start(b, b)

        def outer(o, carry):
            for b in range(_NBUF):
                h = o * _NBUF + b
                g_wait(b)
                o_start(h, b)
                pb = (b - 1) % _NBUF
                nxt = h + _NBUF - 1
                if b == 0:
                    @pl.when(o > 0)
                    def _():
                        o_wait(pb)

                    g_start(nxt, pb)
                else:
                    o_wait(pb)

                    @pl.when(o < n_outer - 1)
                    def _():
                        g_start(nxt, pb)

            return carry

        lax.fori_loop(0, n_outer, outer, 0)

        o_wait(_NBUF - 1)

    return gather_kernel


def kernel(ids, table):
    batch, hist = ids.shape
    ids_t = ids.T.astype(jnp.int32)
    out = _make_gather(batch, hist)(ids_t, table)
    return out.reshape(hist, batch, _D).transpose(1, 0, 2)
